# baseline (speedup 1.0000x reference)
import functools
import jax
import jax.numpy as jnp
from jax import lax
from jax.experimental import pallas as pl
from jax.experimental.pallas import tpu as pltpu
from jax.experimental.pallas import tpu_sc as plsc

_MAX_LEN = 2048
_BATCH = 16
_NUM_AA = 21
_ROWS = _NUM_AA * _BATCH
_HALF = _MAX_LEN // 2


@functools.partial(
    pl.kernel,
    mesh=plsc.VectorSubcoreMesh(core_axis_name="c", subcore_axis_name="s"),
    out_type=jax.ShapeDtypeStruct((_ROWS, _MAX_LEN), jnp.float32),
    scratch_types=[
        pltpu.VMEM((_HALF,), jnp.int32),
        pltpu.VMEM((16,), jnp.int32),
        pltpu.VMEM((_NUM_AA, _HALF), jnp.float32),
        pltpu.SemaphoreType.DMA,
    ],
)
def _sc_onehot(dataT_hbm, lenrep_hbm, out_hbm, data_v, len_v, out_v, sem):
    c = lax.axis_index("c")
    s = lax.axis_index("s")
    t0 = c * _HALF
    pltpu.sync_copy(dataT_hbm.at[s, pl.ds(t0, _HALF)], data_v)
    pltpu.sync_copy(lenrep_hbm.at[s], len_v)
    lenv = len_v[...]

    def body(i, carry):
        d = data_v[pl.ds(i * 16, 16)]
        t = t0 + i * 16 + lax.iota(jnp.int32, 16)
        q = jnp.where(t < lenv, d, _NUM_AA)
        for a in range(_NUM_AA):
            out_v[a, pl.ds(i * 16, 16)] = jnp.where(q == a, 1.0, 0.0)
        return carry

    lax.fori_loop(0, _HALF // 16, body, 0)

    copies = [
        pltpu.async_copy(
            out_v.at[pl.ds(a, 1), :],
            out_hbm.at[pl.ds(a * _BATCH + s, 1), pl.ds(t0, _HALF)],
            sem,
        )
        for a in range(_NUM_AA)
    ]
    for cp in copies:
        cp.wait()


def kernel(data, lengths, embed_init):
    del embed_init
    dataT = jnp.swapaxes(data, 0, 1)
    lenrep = jnp.broadcast_to(
        lengths.astype(jnp.int32).reshape(_BATCH, 1), (_BATCH, 16)
    )
    out_phys = _sc_onehot(dataT, lenrep)
    return jnp.transpose(out_phys.reshape(_NUM_AA, _BATCH, _MAX_LEN), (2, 1, 0))
